# baseline (speedup 1.0000x reference)
import functools

import jax
import jax.numpy as jnp
from jax import lax
from jax.experimental import pallas as pl
from jax.experimental.pallas import tpu as pltpu, tpu_sc as plsc

VOCAB = 1000000
D = 64
B = 4096 * 200
NC, NS = 2, 16
NW = NC * NS
B_PER_W = B // NW
CHUNK = 128
NCHUNK = B_PER_W // CHUNK

_mesh = plsc.VectorSubcoreMesh(
    core_axis_name="c", subcore_axis_name="s", num_cores=NC, num_subcores=NS
)


@functools.partial(
    pl.kernel,
    out_type=jax.ShapeDtypeStruct((B, D), jnp.float32),
    mesh=_mesh,
    compiler_params=pltpu.CompilerParams(use_tc_tiling_on_sc=False),
    scratch_types=[
        pltpu.VMEM((B_PER_W,), jnp.int32),
        pltpu.VMEM((CHUNK, D), jnp.float32),
        pltpu.SemaphoreType.DMA,
    ],
)
def _emb_lookup(idx_hbm, table_hbm, out_hbm, idx_v, rows_v, sem):
    wid = lax.axis_index("s") * NC + lax.axis_index("c")
    base = wid * B_PER_W
    pltpu.sync_copy(idx_hbm.at[pl.ds(base, B_PER_W)], idx_v)

    def body(i, _):
        idx_sl = idx_v.at[pl.ds(i * CHUNK, CHUNK)]
        pltpu.async_copy(table_hbm.at[idx_sl], rows_v, sem).wait()
        pltpu.sync_copy(rows_v, out_hbm.at[pl.ds(base + i * CHUNK, CHUNK)])
        return _

    lax.fori_loop(0, NCHUNK, body, None)


def kernel(x, table):
    out = _emb_lookup(x.reshape(-1), table)
    return out.reshape(x.shape + (D,))


# speedup vs baseline: 1.1085x; 1.1085x over previous
import functools

import jax
import jax.numpy as jnp
from jax import lax
from jax.experimental import pallas as pl
from jax.experimental.pallas import tpu as pltpu, tpu_sc as plsc

VOCAB = 1000000
D = 64
B = 4096 * 200
NC, NS = 2, 16
NW = NC * NS
B_PER_W = B // NW
CHUNK = 128
NCHUNK = B_PER_W // CHUNK
K = 4
G = NCHUNK // K
R = 3

_mesh = plsc.VectorSubcoreMesh(
    core_axis_name="c", subcore_axis_name="s", num_cores=NC, num_subcores=NS
)


@functools.partial(
    pl.kernel,
    out_type=jax.ShapeDtypeStruct((B, D), jnp.float32),
    mesh=_mesh,
    compiler_params=pltpu.CompilerParams(use_tc_tiling_on_sc=False),
    scratch_types=[
        pltpu.VMEM((B_PER_W,), jnp.int32),
        pltpu.VMEM((R, K, CHUNK, D), jnp.float32),
        pltpu.SemaphoreType.DMA,
        pltpu.SemaphoreType.DMA,
        pltpu.SemaphoreType.DMA,
        pltpu.SemaphoreType.DMA,
        pltpu.SemaphoreType.DMA,
        pltpu.SemaphoreType.DMA,
    ],
)
def _emb_lookup(idx_hbm, table_hbm, out_hbm, idx_v, rows_v, g0, g1, g2, s0, s1, s2):
    wid = lax.axis_index("s") * NC + lax.axis_index("c")
    base = wid * B_PER_W
    gsem = (g0, g1, g2)
    ssem = (s0, s1, s2)

    pltpu.sync_copy(idx_hbm.at[pl.ds(base, B_PER_W)], idx_v)

    def gather_desc(gi, r, b):
        idx_sl = idx_v.at[pl.ds((gi * K + b) * CHUNK, CHUNK)]
        return pltpu.make_async_copy(table_hbm.at[idx_sl], rows_v.at[r, b], gsem[r])

    def store_desc(gi, r, b):
        out_sl = out_hbm.at[pl.ds(base + (gi * K + b) * CHUNK, CHUNK)]
        return pltpu.make_async_copy(rows_v.at[r, b], out_sl, ssem[r])

    def fire_g(gi, r):
        for b in range(K):
            gather_desc(gi, r, b).start()

    def drain_g(gi, r):
        for b in range(K):
            gather_desc(gi, r, b).wait()

    def fire_s(gi, r):
        for b in range(K):
            store_desc(gi, r, b).start()

    def drain_s(gi, r):
        for b in range(K):
            store_desc(gi, r, b).wait()

    fire_g(0, 0)
    fire_g(1, 1)
    drain_g(0, 0)
    fire_s(0, 0)
    fire_g(2, 2)
    drain_g(1, 1)
    fire_s(1, 1)

    @pl.loop(2, 47, step=3)
    def _steady(i):
        for p in range(3):
            g = i + p
            r = (2 + p) % R
            rn = (r + 1) % R
            drain_s(g - 2, rn)
            fire_g(g + 1, rn)
            drain_g(g, r)
            fire_s(g, r)

    drain_s(45, 0)
    fire_g(48, 0)
    drain_g(47, 2)
    fire_s(47, 2)

    drain_s(46, 1)
    fire_g(49, 1)
    drain_g(48, 0)
    fire_s(48, 0)

    drain_s(47, 2)
    drain_g(49, 1)
    fire_s(49, 1)

    drain_s(48, 0)
    drain_s(49, 1)


def kernel(x, table):
    out = _emb_lookup(x.reshape(-1), table)
    return out.reshape(x.shape + (D,))


# speedup vs baseline: 1.1112x; 1.0024x over previous
import functools

import jax
import jax.numpy as jnp
from jax import lax
from jax.experimental import pallas as pl
from jax.experimental.pallas import tpu as pltpu, tpu_sc as plsc

VOCAB = 1000000
D = 64
B = 4096 * 200
NC, NS = 2, 16
NW = NC * NS
B_PER_W = B // NW
CHUNK = 128
NCHUNK = B_PER_W // CHUNK
K = 4
G = NCHUNK // K
R = 3

_mesh = plsc.VectorSubcoreMesh(
    core_axis_name="c", subcore_axis_name="s", num_cores=NC, num_subcores=NS
)


@functools.partial(
    pl.kernel,
    out_type=jax.ShapeDtypeStruct((B, D), jnp.float32),
    mesh=_mesh,
    compiler_params=pltpu.CompilerParams(
        use_tc_tiling_on_sc=False,
        has_side_effects=pltpu.SideEffectType.PURE,
    ),
    scratch_types=[
        pltpu.VMEM((B_PER_W,), jnp.int32),
        pltpu.VMEM((R, K, CHUNK, D), jnp.float32),
        pltpu.SemaphoreType.DMA,
        pltpu.SemaphoreType.DMA,
        pltpu.SemaphoreType.DMA,
        pltpu.SemaphoreType.DMA,
        pltpu.SemaphoreType.DMA,
        pltpu.SemaphoreType.DMA,
    ],
)
def _emb_lookup(idx_hbm, table_hbm, out_hbm, idx_v, rows_v, g0, g1, g2, s0, s1, s2):
    wid = lax.axis_index("s") * NC + lax.axis_index("c")
    base = wid * B_PER_W
    gsem = (g0, g1, g2)
    ssem = (s0, s1, s2)

    table_rows = table_hbm
    out_rows = out_hbm

    pltpu.sync_copy(idx_hbm.at[pl.ds(base, B_PER_W)], idx_v)

    def gather_desc(gi, r, b):
        idx_sl = idx_v.at[pl.ds((gi * K + b) * CHUNK, CHUNK)]
        return pltpu.make_async_copy(table_rows.at[idx_sl], rows_v.at[r, b], gsem[r])

    def store_desc(gi, r, b):
        out_sl = out_rows.at[pl.ds(base + (gi * K + b) * CHUNK, CHUNK)]
        return pltpu.make_async_copy(rows_v.at[r, b], out_sl, ssem[r])

    def fire_g(gi, r):
        for b in range(K):
            gather_desc(gi, r, b).start()

    def drain_g(gi, r):
        for b in range(K):
            gather_desc(gi, r, b).wait()

    def fire_s(gi, r):
        for b in range(K):
            store_desc(gi, r, b).start()

    def drain_s(gi, r):
        for b in range(K):
            store_desc(gi, r, b).wait()

    fire_g(0, 0)
    fire_g(1, 1)
    drain_g(0, 0)
    fire_s(0, 0)
    fire_g(2, 2)
    drain_g(1, 1)
    fire_s(1, 1)

    @pl.loop(2, 47, step=3)
    def _steady(i):
        for p in range(3):
            g = i + p
            r = (2 + p) % R
            rn = (r + 1) % R
            drain_s(g - 2, rn)
            fire_g(g + 1, rn)
            drain_g(g, r)
            fire_s(g, r)

    drain_s(45, 0)
    fire_g(48, 0)
    drain_g(47, 2)
    fire_s(47, 2)

    drain_s(46, 1)
    fire_g(49, 1)
    drain_g(48, 0)
    fire_s(48, 0)

    drain_s(47, 2)
    drain_g(49, 1)
    fire_s(49, 1)

    drain_s(48, 0)
    drain_s(49, 1)


def kernel(x, table):
    out = _emb_lookup(x.reshape(-1), table)
    return out.reshape(x.shape + (D,))
